# baseline (speedup 1.0000x reference)
import functools

import jax
import jax.numpy as jnp
from jax import lax
from jax.experimental import pallas as pl
from jax.experimental.pallas import tpu as pltpu
from jax.experimental.pallas import tpu_sc as plsc

N = 50000
EMB = 64
HW = 32
NP = 51200
NPAD = 51200
TRASH = 51100
NTILES = 16
RPT = NPAD // NTILES
CH = 128
GROUP = 8
NGROUPS = 49
CPT = GROUP * NGROUPS
E_PAD = NTILES * CPT * CH
EHALF = E_PAD // 2
G8 = 7
NG8 = EHALF // (NTILES * CH * G8)
BLK = 3200

_mesh = plsc.VectorSubcoreMesh(core_axis_name="c", subcore_axis_name="s")


def _sc_segsum64(table, srcidx, dst2d, zrows):
    @functools.partial(
        pl.kernel,
        mesh=_mesh,
        out_type=jax.ShapeDtypeStruct((NPAD, 128), jnp.float32),
        compiler_params=pltpu.CompilerParams(use_tc_tiling_on_sc=False, needs_layout_passes=False),
        scratch_types=[
            pltpu.VMEM_SHARED((NPAD, HW), jnp.float32),
            pltpu.VMEM((GROUP * CH,), jnp.int32),
            pltpu.VMEM((GROUP, CH), jnp.int32),
            pltpu.VMEM((3, 2 * CH, HW), jnp.float32),
            [pltpu.SemaphoreType.DMA] * 3,
            [pltpu.SemaphoreType.DMA] * 2,
        ],
    )
    def k(tab_hbm, src_hbm, dst_hbm, z_hbm, out_hbm,
          acc, gsrc, gdst, rows, gsems, ssems):
        c = lax.axis_index("c")
        t = lax.axis_index("s")

        pltpu.sync_copy(z_hbm, acc.at[pl.ds(t * RPT, RPT)])
        plsc.subcore_barrier()

        tile_edge = t * CPT * CH
        tile_drow = t * CPT

        def gather(j):
            return pltpu.async_copy(
                tab_hbm.at[gsrc.at[pl.ds(j * 2 * CH, 2 * CH)]],
                rows.at[j % 3], gsems[j % 3])

        NG2 = GROUP // 2

        def group_body(g, carry):
            eoff = tile_edge + g * (GROUP * CH)
            pltpu.sync_copy(src_hbm.at[pl.ds(c * E_PAD + eoff, GROUP * CH)],
                            gsrc)
            pltpu.sync_copy(dst_hbm.at[pl.ds(tile_drow + g * GROUP, GROUP)],
                            gdst)
            gs = [gather(0), gather(1)]
            ss = [None] * NG2
            for j in range(NG2):
                if j >= 1:
                    for s in ss[j - 1]:
                        s.wait()
                if j + 2 < NG2:
                    gs.append(gather(j + 2))
                gs[j].wait()
                ss[j] = [
                    pltpu.async_copy(rows.at[j % 3, pl.ds(q * CH, CH)],
                                     acc.at[gdst.at[2 * j + q]],
                                     ssems[q], add=True)
                    for q in range(2)]
            for s in ss[NG2 - 1]:
                s.wait()
            return carry

        lax.fori_loop(0, NGROUPS, group_body, 0)
        plsc.subcore_barrier()
        pltpu.sync_copy(acc.at[pl.ds(t * RPT, RPT)],
                        out_hbm.at[pl.ds(t * RPT, RPT), pl.ds(c * HW, HW)])

    return k(table, srcidx, dst2d, zrows)


def _sc_segsum8(c0, c1, c2, c3, dst2d, zrows):
    @functools.partial(
        pl.kernel,
        mesh=_mesh,
        out_type=jax.ShapeDtypeStruct((NPAD, 128), jnp.float32),
        compiler_params=pltpu.CompilerParams(use_tc_tiling_on_sc=False, needs_layout_passes=False),
        scratch_types=[
            pltpu.VMEM_SHARED((NPAD, 8), jnp.float32),
            pltpu.VMEM((4, G8 * CH), jnp.float32),
            pltpu.VMEM((G8, CH), jnp.int32),
            pltpu.VMEM((CH, 8), jnp.float32),
        ],
    )
    def k(c0_hbm, c1_hbm, c2_hbm, c3_hbm, dst_hbm, z_hbm, out_hbm,
          acc, cols, gdst, rows8):
        c = lax.axis_index("c")
        t = lax.axis_index("s")

        pltpu.sync_copy(z_hbm, acc.at[pl.ds(t * RPT, RPT)])

        lanes = lax.iota(jnp.int32, 16)

        def init_body(kk, carry):
            f = kk * 16 + lanes
            v = jnp.where((f & 7) == 4, 1.0, 0.0).astype(jnp.float32)
            plsc.store_scatter(rows8, [f >> 3, f & 7], v)
            return carry

        lax.fori_loop(0, CH * 8 // 16, init_body, 0)
        plsc.subcore_barrier()

        tile_edge = c * EHALF + t * NG8 * G8 * CH
        tile_drow = tile_edge // CH
        chbms = [c0_hbm, c1_hbm, c2_hbm, c3_hbm]

        def group_body(g, carry):
            eoff = tile_edge + g * (G8 * CH)
            for ci in range(4):
                pltpu.sync_copy(chbms[ci].at[pl.ds(eoff, G8 * CH)],
                                cols.at[ci])
            pltpu.sync_copy(dst_hbm.at[pl.ds(tile_drow + g * G8, G8)], gdst)
            for j in range(G8):
                def ileave(q, carry2, _j=j):
                    e = q * 16 + lanes
                    for ci in range(4):
                        v = cols[ci, pl.ds(_j * CH + q * 16, 16)]
                        plsc.store_scatter(
                            rows8, [e, jnp.full((16,), ci, jnp.int32)], v)
                    return carry2
                lax.fori_loop(0, CH // 16, ileave, 0)
                pltpu.sync_copy(rows8, acc.at[gdst.at[j]], add=True)
            return carry

        lax.fori_loop(0, NG8, group_body, 0)
        plsc.subcore_barrier()
        pltpu.sync_copy(acc.at[pl.ds(t * RPT, RPT)],
                        out_hbm.at[pl.ds(t * RPT, RPT), pl.ds(c * 8, 8)])

    return k(c0, c1, c2, c3, dst2d, zrows)


def _mm(a, b):
    return jnp.dot(a, b, preferred_element_type=jnp.float32)


_W128 = lambda i: (i, 0)
_W0 = lambda i: (0, 0)


def _embed(x8, Wn8, bn8):
    def body(x_ref, w_ref, b_ref, o_ref):
        r = jnp.maximum(_mm(x_ref[...], w_ref[...]) + b_ref[0:1, :], 0.0)
        o_ref[...] = jnp.concatenate(
            [r, jnp.zeros((BLK, 128 - EMB), jnp.float32)], axis=1)

    return pl.pallas_call(
        body,
        grid=(NP // BLK,),
        in_specs=[pl.BlockSpec((BLK, 8), _W128),
                  pl.BlockSpec((8, EMB), _W0),
                  pl.BlockSpec((8, EMB), _W0)],
        out_specs=pl.BlockSpec((BLK, 128), _W128),
        out_shape=jax.ShapeDtypeStruct((NP, 128), jnp.float32),
    )(x8, Wn8, bn8)


def _update(h128, p128, s128, Wed8, We_i, be_i8, Wu_i, bu_i8):
    def body(h_ref, p_ref, s_ref, wed_ref, we_ref, be_ref, wu_ref, bu_ref,
             o_ref):
        sc8 = s_ref[:, 0:8] + s_ref[:, 8:16]
        cnt = sc8[:, 4:5]
        sea = _mm(sc8, wed_ref[...])
        agg = (_mm(p_ref[:, 0:EMB], we_ref[0:EMB, :])
               + _mm(sea, we_ref[EMB:2 * EMB, :])
               + cnt * be_ref[0:1, :]) / jnp.maximum(cnt, 1.0)
        o = (_mm(h_ref[:, 0:EMB], wu_ref[0:EMB, :])
             + _mm(agg, wu_ref[EMB:2 * EMB, :]) + bu_ref[0:1, :])
        o = jnp.maximum(o, 0.0)
        o_ref[...] = jnp.concatenate(
            [o, jnp.zeros((BLK, 128 - EMB), jnp.float32)], axis=1)

    return pl.pallas_call(
        body,
        grid=(NP // BLK,),
        in_specs=[pl.BlockSpec((BLK, 128), _W128),
                  pl.BlockSpec((BLK, 128), _W128),
                  pl.BlockSpec((BLK, 128), _W128),
                  pl.BlockSpec((8, EMB), _W0),
                  pl.BlockSpec((2 * EMB, EMB), _W0),
                  pl.BlockSpec((8, EMB), _W0),
                  pl.BlockSpec((2 * EMB, EMB), _W0),
                  pl.BlockSpec((8, EMB), _W0)],
        out_specs=pl.BlockSpec((BLK, 128), _W128),
        out_shape=jax.ShapeDtypeStruct((NP, 128), jnp.float32),
    )(h128, p128, s128, Wed8, We_i, be_i8, Wu_i, bu_i8)


def _decode(h128, Wd8, bd8):
    def body(h_ref, w_ref, b_ref, o_ref):
        o_ref[...] = _mm(h_ref[:, 0:EMB], w_ref[...]) + b_ref[0:1, :]

    return pl.pallas_call(
        body,
        grid=(NP // BLK,),
        in_specs=[pl.BlockSpec((BLK, 128), _W128),
                  pl.BlockSpec((EMB, 8), _W0),
                  pl.BlockSpec((8, 8), _W0)],
        out_specs=pl.BlockSpec((BLK, 8), _W128),
        out_shape=jax.ShapeDtypeStruct((NP, 8), jnp.float32),
    )(h128, Wd8, bd8)


def kernel(x, edge_attr, edge_index, Wn, bn, Wed, bed, We, be, Wu, bu, Wd, bd):
    E = edge_index.shape[1]
    src = edge_index[0]
    dst = edge_index[1]

    src_pad = jnp.concatenate([src, jnp.zeros((E_PAD - E,), jnp.int32)])
    srcidx = jnp.concatenate([4 * src_pad, 4 * src_pad + 1])
    dst2d = jnp.concatenate(
        [dst, jnp.full((E_PAD - E,), TRASH, jnp.int32)]).reshape(-1, CH)
    zpad = jnp.zeros((E_PAD - E,), jnp.float32)
    ecols = [jnp.concatenate([edge_attr[:, i], zpad]) for i in range(4)]

    x8 = jnp.pad(x, ((0, NP - N), (0, 1)))
    Wn8 = jnp.concatenate([Wn, jnp.zeros((1, EMB), jnp.float32)], axis=0)
    bn8 = jnp.broadcast_to(bn[None, :], (8, EMB))
    Wed8 = jnp.concatenate(
        [Wed, bed[None, :], jnp.zeros((3, EMB), jnp.float32)], axis=0)
    Wd8 = jnp.concatenate([Wd, jnp.zeros((EMB, 5), jnp.float32)], axis=1)
    bd8 = jnp.broadcast_to(
        jnp.concatenate([bd, jnp.zeros((5,), jnp.float32)])[None, :], (8, 8))

    z32 = jnp.zeros((RPT, HW), jnp.float32)
    z8 = jnp.zeros((RPT, 8), jnp.float32)

    s128 = _sc_segsum8(*ecols, dst2d, z8)

    h128 = _embed(x8, Wn8, bn8)
    for i in range(6):
        table = h128.reshape(4 * NP, HW)
        p128 = _sc_segsum64(table, srcidx, dst2d, z32)
        h128 = _update(h128, p128, s128, Wed8, We[i],
                       jnp.broadcast_to(be[i][None, :], (8, EMB)),
                       Wu[i],
                       jnp.broadcast_to(bu[i][None, :], (8, EMB)))
    return _decode(h128, Wd8, bd8)[:N, :3]


# speedup vs baseline: 1.1774x; 1.1774x over previous
import functools

import jax
import jax.numpy as jnp
from jax import lax
from jax.experimental import pallas as pl
from jax.experimental.pallas import tpu as pltpu
from jax.experimental.pallas import tpu_sc as plsc

N = 50000
EMB = 64
HW = 32
NP = 51200
NPAD = 51200
TRASH = 51100
NTILES = 16
RPT = NPAD // NTILES
CH = 128
GROUP = 14
NGROUPS = 28
CPT = GROUP * NGROUPS
E_PAD = NTILES * CPT * CH
EHALF = E_PAD // 2
G8 = 7
NG8 = EHALF // (NTILES * CH * G8)
BLK = 3200

_mesh = plsc.VectorSubcoreMesh(core_axis_name="c", subcore_axis_name="s")


def _sc_segsum64(table, srcidx, dst2d, zrows):
    @functools.partial(
        pl.kernel,
        mesh=_mesh,
        out_type=jax.ShapeDtypeStruct((NPAD, 128), jnp.float32),
        compiler_params=pltpu.CompilerParams(use_tc_tiling_on_sc=False, needs_layout_passes=False),
        scratch_types=[
            pltpu.VMEM_SHARED((NPAD, HW), jnp.float32),
            pltpu.VMEM((GROUP * CH,), jnp.int32),
            pltpu.VMEM((GROUP, CH), jnp.int32),
            pltpu.VMEM((6, CH, HW), jnp.float32),
            [pltpu.SemaphoreType.DMA] * 6,
            [pltpu.SemaphoreType.DMA] * 3,
        ],
    )
    def k(tab_hbm, src_hbm, dst_hbm, z_hbm, out_hbm,
          acc, gsrc, gdst, rows, gsems, ssems):
        c = lax.axis_index("c")
        t = lax.axis_index("s")

        pltpu.sync_copy(z_hbm, acc.at[pl.ds(t * RPT, RPT)])
        plsc.subcore_barrier()

        tile_edge = t * CPT * CH
        tile_drow = t * CPT

        def gather(j):
            return pltpu.async_copy(
                tab_hbm.at[gsrc.at[pl.ds(j * CH, CH)]],
                rows.at[j % 6], gsems[j % 6])

        def group_body(g, carry):
            eoff = tile_edge + g * (GROUP * CH)
            pltpu.sync_copy(src_hbm.at[pl.ds(c * E_PAD + eoff, GROUP * CH)],
                            gsrc)
            pltpu.sync_copy(dst_hbm.at[pl.ds(tile_drow + g * GROUP, GROUP)],
                            gdst)
            gs = [gather(0), gather(1), gather(2)]
            ss = [None] * GROUP
            for j in range(GROUP):
                if j >= 3:
                    ss[j - 3].wait()
                if j + 3 < GROUP:
                    gs.append(gather(j + 3))
                gs[j].wait()
                ss[j] = pltpu.async_copy(
                    rows.at[j % 6], acc.at[gdst.at[j]],
                    ssems[j % 3], add=True)
            for j in range(GROUP - 3, GROUP):
                ss[j].wait()
            return carry

        lax.fori_loop(0, NGROUPS, group_body, 0)
        plsc.subcore_barrier()
        pltpu.sync_copy(acc.at[pl.ds(t * RPT, RPT)],
                        out_hbm.at[pl.ds(t * RPT, RPT), pl.ds(c * HW, HW)])

    return k(table, srcidx, dst2d, zrows)


def _sc_segsum8(c0, c1, c2, c3, dst2d, zrows):
    @functools.partial(
        pl.kernel,
        mesh=_mesh,
        out_type=jax.ShapeDtypeStruct((NPAD, 128), jnp.float32),
        compiler_params=pltpu.CompilerParams(use_tc_tiling_on_sc=False, needs_layout_passes=False),
        scratch_types=[
            pltpu.VMEM_SHARED((NPAD, 8), jnp.float32),
            pltpu.VMEM((4, G8 * CH), jnp.float32),
            pltpu.VMEM((G8, CH), jnp.int32),
            pltpu.VMEM((CH, 8), jnp.float32),
        ],
    )
    def k(c0_hbm, c1_hbm, c2_hbm, c3_hbm, dst_hbm, z_hbm, out_hbm,
          acc, cols, gdst, rows8):
        c = lax.axis_index("c")
        t = lax.axis_index("s")

        pltpu.sync_copy(z_hbm, acc.at[pl.ds(t * RPT, RPT)])

        lanes = lax.iota(jnp.int32, 16)

        def init_body(kk, carry):
            f = kk * 16 + lanes
            v = jnp.where((f & 7) == 4, 1.0, 0.0).astype(jnp.float32)
            plsc.store_scatter(rows8, [f >> 3, f & 7], v)
            return carry

        lax.fori_loop(0, CH * 8 // 16, init_body, 0)
        plsc.subcore_barrier()

        tile_edge = c * EHALF + t * NG8 * G8 * CH
        tile_drow = tile_edge // CH
        chbms = [c0_hbm, c1_hbm, c2_hbm, c3_hbm]

        def group_body(g, carry):
            eoff = tile_edge + g * (G8 * CH)
            for ci in range(4):
                pltpu.sync_copy(chbms[ci].at[pl.ds(eoff, G8 * CH)],
                                cols.at[ci])
            pltpu.sync_copy(dst_hbm.at[pl.ds(tile_drow + g * G8, G8)], gdst)
            for j in range(G8):
                def ileave(q, carry2, _j=j):
                    e = q * 16 + lanes
                    for ci in range(4):
                        v = cols[ci, pl.ds(_j * CH + q * 16, 16)]
                        plsc.store_scatter(
                            rows8, [e, jnp.full((16,), ci, jnp.int32)], v)
                    return carry2
                lax.fori_loop(0, CH // 16, ileave, 0)
                pltpu.sync_copy(rows8, acc.at[gdst.at[j]], add=True)
            return carry

        lax.fori_loop(0, NG8, group_body, 0)
        plsc.subcore_barrier()
        pltpu.sync_copy(acc.at[pl.ds(t * RPT, RPT)],
                        out_hbm.at[pl.ds(t * RPT, RPT), pl.ds(c * 8, 8)])

    return k(c0, c1, c2, c3, dst2d, zrows)


def _mm(a, b):
    return jnp.dot(a, b, preferred_element_type=jnp.float32)


_W128 = lambda i: (i, 0)
_W0 = lambda i: (0, 0)


def _embed(x8, Wn8, bn8):
    def body(x_ref, w_ref, b_ref, o_ref):
        r = jnp.maximum(_mm(x_ref[...], w_ref[...]) + b_ref[0:1, :], 0.0)
        o_ref[...] = jnp.concatenate(
            [r, jnp.zeros((BLK, 128 - EMB), jnp.float32)], axis=1)

    return pl.pallas_call(
        body,
        grid=(NP // BLK,),
        in_specs=[pl.BlockSpec((BLK, 8), _W128),
                  pl.BlockSpec((8, EMB), _W0),
                  pl.BlockSpec((8, EMB), _W0)],
        out_specs=pl.BlockSpec((BLK, 128), _W128),
        out_shape=jax.ShapeDtypeStruct((NP, 128), jnp.float32),
    )(x8, Wn8, bn8)


def _update(h128, p128, s128, Wed8, We_i, be_i8, Wu_i, bu_i8):
    def body(h_ref, p_ref, s_ref, wed_ref, we_ref, be_ref, wu_ref, bu_ref,
             o_ref):
        sc8 = s_ref[:, 0:8] + s_ref[:, 8:16]
        cnt = sc8[:, 4:5]
        sea = _mm(sc8, wed_ref[...])
        agg = (_mm(p_ref[:, 0:EMB], we_ref[0:EMB, :])
               + _mm(sea, we_ref[EMB:2 * EMB, :])
               + cnt * be_ref[0:1, :]) / jnp.maximum(cnt, 1.0)
        o = (_mm(h_ref[:, 0:EMB], wu_ref[0:EMB, :])
             + _mm(agg, wu_ref[EMB:2 * EMB, :]) + bu_ref[0:1, :])
        o = jnp.maximum(o, 0.0)
        o_ref[...] = jnp.concatenate(
            [o, jnp.zeros((BLK, 128 - EMB), jnp.float32)], axis=1)

    return pl.pallas_call(
        body,
        grid=(NP // BLK,),
        in_specs=[pl.BlockSpec((BLK, 128), _W128),
                  pl.BlockSpec((BLK, 128), _W128),
                  pl.BlockSpec((BLK, 128), _W128),
                  pl.BlockSpec((8, EMB), _W0),
                  pl.BlockSpec((2 * EMB, EMB), _W0),
                  pl.BlockSpec((8, EMB), _W0),
                  pl.BlockSpec((2 * EMB, EMB), _W0),
                  pl.BlockSpec((8, EMB), _W0)],
        out_specs=pl.BlockSpec((BLK, 128), _W128),
        out_shape=jax.ShapeDtypeStruct((NP, 128), jnp.float32),
    )(h128, p128, s128, Wed8, We_i, be_i8, Wu_i, bu_i8)


def _decode(h128, Wd8, bd8):
    def body(h_ref, w_ref, b_ref, o_ref):
        o_ref[...] = _mm(h_ref[:, 0:EMB], w_ref[...]) + b_ref[0:1, :]

    return pl.pallas_call(
        body,
        grid=(NP // BLK,),
        in_specs=[pl.BlockSpec((BLK, 128), _W128),
                  pl.BlockSpec((EMB, 8), _W0),
                  pl.BlockSpec((8, 8), _W0)],
        out_specs=pl.BlockSpec((BLK, 8), _W128),
        out_shape=jax.ShapeDtypeStruct((NP, 8), jnp.float32),
    )(h128, Wd8, bd8)


def kernel(x, edge_attr, edge_index, Wn, bn, Wed, bed, We, be, Wu, bu, Wd, bd):
    E = edge_index.shape[1]
    src = edge_index[0]
    dst = edge_index[1]

    src_pad = jnp.concatenate([src, jnp.zeros((E_PAD - E,), jnp.int32)])
    srcidx = jnp.concatenate([4 * src_pad, 4 * src_pad + 1])
    dst2d = jnp.concatenate(
        [dst, jnp.full((E_PAD - E,), TRASH, jnp.int32)]).reshape(-1, CH)
    zpad = jnp.zeros((E_PAD - E,), jnp.float32)
    ecols = [jnp.concatenate([edge_attr[:, i], zpad]) for i in range(4)]

    x8 = jnp.pad(x, ((0, NP - N), (0, 1)))
    Wn8 = jnp.concatenate([Wn, jnp.zeros((1, EMB), jnp.float32)], axis=0)
    bn8 = jnp.broadcast_to(bn[None, :], (8, EMB))
    Wed8 = jnp.concatenate(
        [Wed, bed[None, :], jnp.zeros((3, EMB), jnp.float32)], axis=0)
    Wd8 = jnp.concatenate([Wd, jnp.zeros((EMB, 5), jnp.float32)], axis=1)
    bd8 = jnp.broadcast_to(
        jnp.concatenate([bd, jnp.zeros((5,), jnp.float32)])[None, :], (8, 8))

    z32 = jnp.zeros((RPT, HW), jnp.float32)
    z8 = jnp.zeros((RPT, 8), jnp.float32)

    s128 = _sc_segsum8(*ecols, dst2d, z8)

    h128 = _embed(x8, Wn8, bn8)
    for i in range(6):
        table = h128.reshape(4 * NP, HW)
        p128 = _sc_segsum64(table, srcidx, dst2d, z32)
        h128 = _update(h128, p128, s128, Wed8, We[i],
                       jnp.broadcast_to(be[i][None, :], (8, EMB)),
                       Wu[i],
                       jnp.broadcast_to(bu[i][None, :], (8, EMB)))
    return _decode(h128, Wd8, bd8)[:N, :3]


# speedup vs baseline: 1.2451x; 1.0575x over previous
import functools

import jax
import jax.numpy as jnp
from jax import lax
from jax.experimental import pallas as pl
from jax.experimental.pallas import tpu as pltpu
from jax.experimental.pallas import tpu_sc as plsc

N = 50000
EMB = 64
HW = 32
NP = 51200
NPAD = 51200
TRASH = 51100
NTILES = 16
RPT = NPAD // NTILES
CH = 128
GROUP = 28
NGROUPS = 14
CPT = GROUP * NGROUPS
E_PAD = NTILES * CPT * CH
EHALF = E_PAD // 2
G8 = 7
NG8 = EHALF // (NTILES * CH * G8)
BLK = 3200

_mesh = plsc.VectorSubcoreMesh(core_axis_name="c", subcore_axis_name="s")


def _sc_segsum64(table, srcidx, dst2d, zrows):
    @functools.partial(
        pl.kernel,
        mesh=_mesh,
        out_type=jax.ShapeDtypeStruct((NPAD, 128), jnp.float32),
        compiler_params=pltpu.CompilerParams(use_tc_tiling_on_sc=False, needs_layout_passes=False),
        scratch_types=[
            pltpu.VMEM_SHARED((NPAD, HW), jnp.float32),
            pltpu.VMEM((GROUP * CH,), jnp.int32),
            pltpu.VMEM((GROUP, CH), jnp.int32),
            pltpu.VMEM((5, CH, HW), jnp.float32),
            [pltpu.SemaphoreType.DMA] * 5,
            [pltpu.SemaphoreType.DMA] * 2,
        ],
    )
    def k(tab_hbm, src_hbm, dst_hbm, z_hbm, out_hbm,
          acc, gsrc, gdst, rows, gsems, ssems):
        c = lax.axis_index("c")
        t = lax.axis_index("s")

        pltpu.sync_copy(z_hbm, acc.at[pl.ds(t * RPT, RPT)])
        plsc.subcore_barrier()

        tile_edge = t * CPT * CH
        tile_drow = t * CPT

        def gather(j):
            return pltpu.async_copy(
                tab_hbm.at[gsrc.at[pl.ds(j * CH, CH)]],
                rows.at[j % 5], gsems[j % 5])

        def group_body(g, carry):
            eoff = tile_edge + g * (GROUP * CH)
            pltpu.sync_copy(src_hbm.at[pl.ds(c * E_PAD + eoff, GROUP * CH)],
                            gsrc)
            pltpu.sync_copy(dst_hbm.at[pl.ds(tile_drow + g * GROUP, GROUP)],
                            gdst)
            gs = [gather(0), gather(1), gather(2)]
            ss = [None] * GROUP
            for j in range(GROUP):
                if j >= 2:
                    ss[j - 2].wait()
                if j + 3 < GROUP:
                    gs.append(gather(j + 3))
                gs[j].wait()
                ss[j] = pltpu.async_copy(
                    rows.at[j % 5], acc.at[gdst.at[j]],
                    ssems[j % 2], add=True)
            ss[GROUP - 2].wait()
            ss[GROUP - 1].wait()
            return carry

        lax.fori_loop(0, NGROUPS, group_body, 0)
        plsc.subcore_barrier()
        pltpu.sync_copy(acc.at[pl.ds(t * RPT, RPT)],
                        out_hbm.at[pl.ds(t * RPT, RPT), pl.ds(c * HW, HW)])

    return k(table, srcidx, dst2d, zrows)


def _sc_segsum8(c0, c1, c2, c3, dst2d, zrows):
    @functools.partial(
        pl.kernel,
        mesh=_mesh,
        out_type=jax.ShapeDtypeStruct((NPAD, 128), jnp.float32),
        compiler_params=pltpu.CompilerParams(use_tc_tiling_on_sc=False, needs_layout_passes=False),
        scratch_types=[
            pltpu.VMEM_SHARED((NPAD, 8), jnp.float32),
            pltpu.VMEM((4, G8 * CH), jnp.float32),
            pltpu.VMEM((G8, CH), jnp.int32),
            pltpu.VMEM((CH, 8), jnp.float32),
        ],
    )
    def k(c0_hbm, c1_hbm, c2_hbm, c3_hbm, dst_hbm, z_hbm, out_hbm,
          acc, cols, gdst, rows8):
        c = lax.axis_index("c")
        t = lax.axis_index("s")

        pltpu.sync_copy(z_hbm, acc.at[pl.ds(t * RPT, RPT)])

        lanes = lax.iota(jnp.int32, 16)

        def init_body(kk, carry):
            f = kk * 16 + lanes
            v = jnp.where((f & 7) == 4, 1.0, 0.0).astype(jnp.float32)
            plsc.store_scatter(rows8, [f >> 3, f & 7], v)
            return carry

        lax.fori_loop(0, CH * 8 // 16, init_body, 0)
        plsc.subcore_barrier()

        tile_edge = c * EHALF + t * NG8 * G8 * CH
        tile_drow = tile_edge // CH
        chbms = [c0_hbm, c1_hbm, c2_hbm, c3_hbm]

        def group_body(g, carry):
            eoff = tile_edge + g * (G8 * CH)
            for ci in range(4):
                pltpu.sync_copy(chbms[ci].at[pl.ds(eoff, G8 * CH)],
                                cols.at[ci])
            pltpu.sync_copy(dst_hbm.at[pl.ds(tile_drow + g * G8, G8)], gdst)
            for j in range(G8):
                def ileave(q, carry2, _j=j):
                    e = q * 16 + lanes
                    for ci in range(4):
                        v = cols[ci, pl.ds(_j * CH + q * 16, 16)]
                        plsc.store_scatter(
                            rows8, [e, jnp.full((16,), ci, jnp.int32)], v)
                    return carry2
                lax.fori_loop(0, CH // 16, ileave, 0)
                pltpu.sync_copy(rows8, acc.at[gdst.at[j]], add=True)
            return carry

        lax.fori_loop(0, NG8, group_body, 0)
        plsc.subcore_barrier()
        pltpu.sync_copy(acc.at[pl.ds(t * RPT, RPT)],
                        out_hbm.at[pl.ds(t * RPT, RPT), pl.ds(c * 8, 8)])

    return k(c0, c1, c2, c3, dst2d, zrows)


def _mm(a, b):
    return jnp.dot(a, b, preferred_element_type=jnp.float32)


_W128 = lambda i: (i, 0)
_W0 = lambda i: (0, 0)


def _embed(x8, Wn8, bn8):
    def body(x_ref, w_ref, b_ref, o_ref):
        r = jnp.maximum(_mm(x_ref[...], w_ref[...]) + b_ref[0:1, :], 0.0)
        o_ref[...] = jnp.concatenate(
            [r, jnp.zeros((BLK, 128 - EMB), jnp.float32)], axis=1)

    return pl.pallas_call(
        body,
        grid=(NP // BLK,),
        in_specs=[pl.BlockSpec((BLK, 8), _W128),
                  pl.BlockSpec((8, EMB), _W0),
                  pl.BlockSpec((8, EMB), _W0)],
        out_specs=pl.BlockSpec((BLK, 128), _W128),
        out_shape=jax.ShapeDtypeStruct((NP, 128), jnp.float32),
    )(x8, Wn8, bn8)


def _update(h128, p128, s128, Wed8, We_i, be_i8, Wu_i, bu_i8):
    def body(h_ref, p_ref, s_ref, wed_ref, we_ref, be_ref, wu_ref, bu_ref,
             o_ref):
        sc8 = s_ref[:, 0:8] + s_ref[:, 8:16]
        cnt = sc8[:, 4:5]
        sea = _mm(sc8, wed_ref[...])
        agg = (_mm(p_ref[:, 0:EMB], we_ref[0:EMB, :])
               + _mm(sea, we_ref[EMB:2 * EMB, :])
               + cnt * be_ref[0:1, :]) / jnp.maximum(cnt, 1.0)
        o = (_mm(h_ref[:, 0:EMB], wu_ref[0:EMB, :])
             + _mm(agg, wu_ref[EMB:2 * EMB, :]) + bu_ref[0:1, :])
        o = jnp.maximum(o, 0.0)
        o_ref[...] = jnp.concatenate(
            [o, jnp.zeros((BLK, 128 - EMB), jnp.float32)], axis=1)

    return pl.pallas_call(
        body,
        grid=(NP // BLK,),
        in_specs=[pl.BlockSpec((BLK, 128), _W128),
                  pl.BlockSpec((BLK, 128), _W128),
                  pl.BlockSpec((BLK, 128), _W128),
                  pl.BlockSpec((8, EMB), _W0),
                  pl.BlockSpec((2 * EMB, EMB), _W0),
                  pl.BlockSpec((8, EMB), _W0),
                  pl.BlockSpec((2 * EMB, EMB), _W0),
                  pl.BlockSpec((8, EMB), _W0)],
        out_specs=pl.BlockSpec((BLK, 128), _W128),
        out_shape=jax.ShapeDtypeStruct((NP, 128), jnp.float32),
    )(h128, p128, s128, Wed8, We_i, be_i8, Wu_i, bu_i8)


def _decode(h128, Wd8, bd8):
    def body(h_ref, w_ref, b_ref, o_ref):
        o_ref[...] = _mm(h_ref[:, 0:EMB], w_ref[...]) + b_ref[0:1, :]

    return pl.pallas_call(
        body,
        grid=(NP // BLK,),
        in_specs=[pl.BlockSpec((BLK, 128), _W128),
                  pl.BlockSpec((EMB, 8), _W0),
                  pl.BlockSpec((8, 8), _W0)],
        out_specs=pl.BlockSpec((BLK, 8), _W128),
        out_shape=jax.ShapeDtypeStruct((NP, 8), jnp.float32),
    )(h128, Wd8, bd8)


def kernel(x, edge_attr, edge_index, Wn, bn, Wed, bed, We, be, Wu, bu, Wd, bd):
    E = edge_index.shape[1]
    src = edge_index[0]
    dst = edge_index[1]

    src_pad = jnp.concatenate([src, jnp.zeros((E_PAD - E,), jnp.int32)])
    srcidx = jnp.concatenate([4 * src_pad, 4 * src_pad + 1])
    dst2d = jnp.concatenate(
        [dst, jnp.full((E_PAD - E,), TRASH, jnp.int32)]).reshape(-1, CH)
    zpad = jnp.zeros((E_PAD - E,), jnp.float32)
    ecols = [jnp.concatenate([edge_attr[:, i], zpad]) for i in range(4)]

    x8 = jnp.pad(x, ((0, NP - N), (0, 1)))
    Wn8 = jnp.concatenate([Wn, jnp.zeros((1, EMB), jnp.float32)], axis=0)
    bn8 = jnp.broadcast_to(bn[None, :], (8, EMB))
    Wed8 = jnp.concatenate(
        [Wed, bed[None, :], jnp.zeros((3, EMB), jnp.float32)], axis=0)
    Wd8 = jnp.concatenate([Wd, jnp.zeros((EMB, 5), jnp.float32)], axis=1)
    bd8 = jnp.broadcast_to(
        jnp.concatenate([bd, jnp.zeros((5,), jnp.float32)])[None, :], (8, 8))

    z32 = jnp.zeros((RPT, HW), jnp.float32)
    z8 = jnp.zeros((RPT, 8), jnp.float32)

    s128 = _sc_segsum8(*ecols, dst2d, z8)

    h128 = _embed(x8, Wn8, bn8)
    for i in range(6):
        table = h128.reshape(4 * NP, HW)
        p128 = _sc_segsum64(table, srcidx, dst2d, z32)
        h128 = _update(h128, p128, s128, Wed8, We[i],
                       jnp.broadcast_to(be[i][None, :], (8, EMB)),
                       Wu[i],
                       jnp.broadcast_to(bu[i][None, :], (8, EMB)))
    return _decode(h128, Wd8, bd8)[:N, :3]
